# Initial kernel scaffold; baseline (speedup 1.0000x reference)
import jax
import jax.numpy as jnp
from jax.experimental import pallas as pl


def kernel(users, times, locs, app_seq, edge_index, tla_emb, Ws1, Wn1, b1, Ws2, Wn2, b2):
    raise NotImplementedError("write your pallas kernel here")


# speedup vs baseline: 1.9559x; 1.9559x over previous
import functools

import jax
import jax.numpy as jnp
from jax import lax
from jax.experimental import pallas as pl
from jax.experimental.pallas import tpu as pltpu
from jax.experimental.pallas import tpu_sc as plsc

_N_TIMES = 1000
_N_LOCS = 1000
_N_APPS = 8000
_N_NODES = 10000
_DIM = 128
_SEQ = 20

_NP = 10240
_TRASH = 10100
_CH = 128
_NSUB = 16
_NW = 32
_ROWS_PW = _NP // _NSUB


def _sc_aggregate(table, src_all, dst_all, zeros, width):
    epw = src_all.shape[0] // _NW
    nch = epw // _CH
    mesh = plsc.VectorSubcoreMesh(core_axis_name="c", subcore_axis_name="s")

    @functools.partial(
        pl.kernel,
        out_type=jax.ShapeDtypeStruct((_NW, _ROWS_PW, width), jnp.float32),
        mesh=mesh,
        scratch_types=[
            pltpu.VMEM((_CH,), jnp.int32),
            pltpu.VMEM((_CH,), jnp.int32),
            pltpu.VMEM((_CH, width), jnp.float32),
            pltpu.VMEM_SHARED((_NP, width), jnp.float32),
            pltpu.SemaphoreType.DMA,
        ],
    )
    def k(table_h, src_h, dst_h, zeros_h, out_h, src_v, dst_v, rows_v,
          acc_sh, sem):
        c = lax.axis_index("c")
        s = lax.axis_index("s")
        wid = c * _NSUB + s
        pltpu.sync_copy(zeros_h, acc_sh.at[pl.ds(s * _ROWS_PW, _ROWS_PW)])
        plsc.subcore_barrier()
        base = wid * epw

        def body(t, carry):
            off = base + t * _CH
            pltpu.sync_copy(src_h.at[pl.ds(off, _CH)], src_v)
            pltpu.sync_copy(dst_h.at[pl.ds(off, _CH)], dst_v)
            pltpu.async_copy(table_h.at[src_v], rows_v, sem).wait()
            pltpu.sync_copy(rows_v, acc_sh.at[dst_v], add=True)
            return carry

        lax.fori_loop(0, nch, body, 0)
        plsc.subcore_barrier()
        pltpu.sync_copy(acc_sh.at[pl.ds(s * _ROWS_PW, _ROWS_PW)],
                        out_h.at[wid])

    out = k(table, src_all, dst_all, zeros)
    return out.reshape(2, _NP, width)


def _sc_gather(table, idx):
    n = idx.shape[0]
    per_w = n // _NW
    nch = per_w // _CH
    mesh = plsc.VectorSubcoreMesh(core_axis_name="c", subcore_axis_name="s")

    @functools.partial(
        pl.kernel,
        out_type=jax.ShapeDtypeStruct((n, _DIM), jnp.float32),
        mesh=mesh,
        scratch_types=[
            pltpu.VMEM((_CH,), jnp.int32),
            pltpu.VMEM((_CH, _DIM), jnp.float32),
            pltpu.SemaphoreType.DMA,
        ],
    )
    def k(table_h, idx_h, out_h, idx_v, rows_v, sem):
        c = lax.axis_index("c")
        s = lax.axis_index("s")
        base = (c * _NSUB + s) * per_w

        def body(t, carry):
            off = base + t * _CH
            pltpu.sync_copy(idx_h.at[pl.ds(off, _CH)], idx_v)
            pltpu.async_copy(table_h.at[idx_v], rows_v, sem).wait()
            pltpu.sync_copy(rows_v, out_h.at[pl.ds(off, _CH)])
            return carry

        lax.fori_loop(0, nch, body, 0)

    return k(table, idx)


_RB = 1024


def _mm_body(x_ref, sp_ref, sn_ref, dp_ref, dn_ref, ws_ref, wn_ref, b_ref,
             hap_ref, han_ref, hb_ref):
    x = x_ref[...]
    ws = ws_ref[...]
    wn = wn_ref[...]
    b = b_ref[...]
    s = lax.dot_general(x, ws, (((1,), (1,)), ((), ())),
                        preferred_element_type=jnp.float32)
    np_ = sp_ref[...] / jnp.maximum(dp_ref[...][:, 0:1], 1.0)
    nn_ = sn_ref[...] / jnp.maximum(dn_ref[...][:, 0:1], 1.0)
    mp = lax.dot_general(np_, wn, (((1,), (1,)), ((), ())),
                         preferred_element_type=jnp.float32)
    mn = lax.dot_general(nn_, wn, (((1,), (1,)), ((), ())),
                         preferred_element_type=jnp.float32)
    hap_ref[...] = jnp.maximum(s + mp + b, 0.0)
    han_ref[...] = jnp.maximum(s + mn + b, 0.0)
    hb_ref[...] = jnp.maximum(s + b, 0.0)


def _tc_sage(x, sum_p, sum_n, deg_p, deg_n, Ws, Wn, b):
    grid = (_NP // _RB,)
    row = pl.BlockSpec((_RB, _DIM), lambda i: (i, 0))
    dcol = pl.BlockSpec((_RB, 16), lambda i: (i, 0))
    full = pl.BlockSpec((_DIM, _DIM), lambda i: (0, 0))
    bspec = pl.BlockSpec((1, _DIM), lambda i: (0, 0))
    out_sh = jax.ShapeDtypeStruct((_NP, _DIM), jnp.float32)
    return pl.pallas_call(
        _mm_body,
        grid=grid,
        in_specs=[row, row, row, dcol, dcol, full, full, bspec],
        out_specs=[row, row, row],
        out_shape=[out_sh, out_sh, out_sh],
    )(x, sum_p, sum_n, deg_p, deg_n, Ws, Wn, b.reshape(1, _DIM))


def _mm2_body(xp_ref, xn_ref, xb_ref, sp_ref, sn_ref, dp_ref, dn_ref,
              ws_ref, wn_ref, b_ref, hap_ref, han_ref, hb_ref):
    ws = ws_ref[...]
    wn = wn_ref[...]
    b = b_ref[...]
    dot = lambda a, w: lax.dot_general(a, w, (((1,), (1,)), ((), ())),
                                       preferred_element_type=jnp.float32)
    np_ = sp_ref[...] / jnp.maximum(dp_ref[...][:, 0:1], 1.0)
    nn_ = sn_ref[...] / jnp.maximum(dn_ref[...][:, 0:1], 1.0)
    hap_ref[...] = jnp.maximum(dot(xp_ref[...], ws) + dot(np_, wn) + b, 0.0)
    han_ref[...] = jnp.maximum(dot(xn_ref[...], ws) + dot(nn_, wn) + b, 0.0)
    hb_ref[...] = jnp.maximum(dot(xb_ref[...], ws) + b, 0.0)


def _tc_sage2(xp, xn, xb, sum_p, sum_n, deg_p, deg_n, Ws, Wn, b):
    grid = (_NP // _RB,)
    row = pl.BlockSpec((_RB, _DIM), lambda i: (i, 0))
    dcol = pl.BlockSpec((_RB, 16), lambda i: (i, 0))
    full = pl.BlockSpec((_DIM, _DIM), lambda i: (0, 0))
    bspec = pl.BlockSpec((1, _DIM), lambda i: (0, 0))
    out_sh = jax.ShapeDtypeStruct((_NP, _DIM), jnp.float32)
    return pl.pallas_call(
        _mm2_body,
        grid=grid,
        in_specs=[row, row, row, row, row, dcol, dcol, full, full, bspec],
        out_specs=[row, row, row],
        out_shape=[out_sh, out_sh, out_sh],
    )(xp, xn, xb, sum_p, sum_n, deg_p, deg_n, Ws, Wn, b.reshape(1, _DIM))


_SB = 128


def _sim_body(g_ref, out_ref):
    h = g_ref[...]
    t = h[:, 0, :]
    l = h[:, 1, :]
    a = h[:, 2:, :]
    nt = jnp.sqrt(jnp.sum(t * t, axis=-1, keepdims=True))
    nl = jnp.sqrt(jnp.sum(l * l, axis=-1, keepdims=True))
    na = jnp.sqrt(jnp.sum(a * a, axis=-1))
    tl = t + l
    ntl = jnp.sqrt(jnp.sum(tl * tl, axis=-1, keepdims=True))
    u = l[:, None, :] + a
    nu = jnp.sqrt(jnp.sum(u * u, axis=-1))
    v = t[:, None, :] + a
    nv = jnp.sqrt(jnp.sum(v * v, axis=-1))
    dt = jnp.sum(t[:, None, :] * u, axis=-1)
    dl = jnp.sum(l[:, None, :] * v, axis=-1)
    da = jnp.sum(a * tl[:, None, :], axis=-1)
    sim_t = jnp.sum(dt / (nt * nu), axis=-1)
    sim_l = jnp.sum(dl / (nl * nv), axis=-1)
    sim_a = jnp.sum(da / (na * ntl), axis=-1)
    out_ref[...] = sim_t + sim_l + sim_a


def _tc_sim(g, nrows):
    grid = (nrows // _SB,)
    return pl.pallas_call(
        _sim_body,
        grid=grid,
        in_specs=[pl.BlockSpec((_SB, _SEQ + 2, _DIM), lambda i: (i, 0, 0))],
        out_specs=pl.BlockSpec((_SB,), lambda i: (i,)),
        out_shape=jax.ShapeDtypeStruct((nrows,), jnp.float32),
    )(g)


def kernel(users, times, locs, app_seq, edge_index, tla_emb,
           Ws1, Wn1, b1, Ws2, Wn2, b2):
    batch = users.shape[0]
    m = batch * (_SEQ + 2)
    nodes_idx = jnp.concatenate(
        [_N_APPS + _N_LOCS + times, _N_APPS + locs, app_seq],
        axis=1).reshape(-1)
    nk = jax.random.key(42)
    nks = jax.random.split(nk, 4)
    neg_t = jax.random.randint(nks[1], (batch, 1), 0, _N_TIMES)
    neg_l = jax.random.randint(nks[2], (batch, 1), 0, _N_LOCS)
    neg_a = jax.random.randint(nks[3], (batch, _SEQ), 0, _N_APPS)
    neg_nodes_idx = jnp.concatenate(
        [_N_APPS + _N_LOCS + neg_t, _N_APPS + neg_l, neg_a],
        axis=1).reshape(-1)

    ar = jnp.arange(m, dtype=jnp.int32)
    lp_pos = jnp.full((_N_NODES,), -1, jnp.int32).at[nodes_idx].max(ar)
    lp_neg = jnp.full((_N_NODES,), -1, jnp.int32).at[neg_nodes_idx].max(ar)
    act_pos = lp_pos >= 0
    act_neg = lp_neg >= 0

    src, dst = edge_index[0], edge_index[1]
    e = src.shape[0]
    ep = ((e * 2 // (_NW * _CH)) + (1 if (e * 2) % (_NW * _CH) else 0))
    ep = (ep * _NW * _CH) // 2
    pad = ep - e
    src_p = jnp.concatenate([src.astype(jnp.int32),
                             jnp.zeros((pad,), jnp.int32)])
    dst_p = jnp.concatenate([dst.astype(jnp.int32),
                             jnp.full((pad,), _TRASH, jnp.int32)])
    dstp_pos = jnp.where(act_pos[src_p], dst_p, _TRASH)
    dstp_neg = jnp.where(act_neg[src_p], dst_p, _TRASH)
    dst_all = jnp.concatenate([dstp_pos, dstp_neg])
    src_all_a = jnp.concatenate([src_p, src_p])
    src_all_b = jnp.concatenate([src_p, src_p + _NP])

    x_pad = jnp.zeros((_NP, _DIM), jnp.float32).at[:_N_NODES].set(tla_emb)

    deg_p = jnp.broadcast_to(
        jnp.zeros((_NP,), jnp.float32).at[dstp_pos].add(1.0).reshape(-1, 1),
        (_NP, 16))
    deg_n = jnp.broadcast_to(
        jnp.zeros((_NP,), jnp.float32).at[dstp_neg].add(1.0).reshape(-1, 1),
        (_NP, 16))

    acc_a = _sc_aggregate(x_pad, src_all_a, dst_all,
                          jnp.zeros((_ROWS_PW, _DIM), jnp.float32), _DIM)
    sum1_p, sum1_n = acc_a[0], acc_a[1]
    h1a_p, h1a_n, h1b = _tc_sage(x_pad, sum1_p, sum1_n, deg_p, deg_n,
                                 Ws1, Wn1, b1)

    tb = jnp.concatenate([h1a_p, h1a_n], axis=0)
    acc_b = _sc_aggregate(tb, src_all_b, dst_all,
                          jnp.zeros((_ROWS_PW, _DIM), jnp.float32), _DIM)
    sum2_p, sum2_n = acc_b[0], acc_b[1]

    h2a_p, h2a_n, h2b = _tc_sage2(h1a_p, h1a_n, h1b, sum2_p, sum2_n,
                                  deg_p, deg_n, Ws2, Wn2, b2)

    tf = jnp.concatenate([h2a_p, h2a_n, h2b], axis=0)
    is_last_p = lp_pos[nodes_idx] == ar
    is_last_n = lp_neg[neg_nodes_idx] == ar
    idx_pos = jnp.where(is_last_p, nodes_idx, nodes_idx + 2 * _NP)
    idx_neg = jnp.where(is_last_n, neg_nodes_idx + _NP,
                        neg_nodes_idx + 2 * _NP)
    idx_all = jnp.concatenate([idx_pos, idx_neg]).astype(jnp.int32)

    g = _sc_gather(tf, idx_all)
    loss = _tc_sim(g.reshape(2 * batch, _SEQ + 2, _DIM), 2 * batch)
    return loss[:batch], loss[batch:]


# speedup vs baseline: 7.1438x; 3.6524x over previous
import functools

import jax
import jax.numpy as jnp
from jax import lax
from jax.experimental import pallas as pl
from jax.experimental.pallas import tpu as pltpu
from jax.experimental.pallas import tpu_sc as plsc

_N_TIMES = 1000
_N_LOCS = 1000
_N_APPS = 8000
_N_NODES = 10000
_DIM = 128
_SEQ = 20

_NP = 10240
_TRASH = 10100
_CH = 128
_NSUB = 16
_NW = 32
_ROWS_PW = _NP // _NSUB


def _sc_aggregate(table_p, table_n, src, dst, act_p, act_n, with_deg):
    ep = src.shape[0]
    epw = ep // _NSUB
    nch = epw // _CH
    mesh = plsc.VectorSubcoreMesh(core_axis_name="c", subcore_axis_name="s")

    out_type = [jax.ShapeDtypeStruct((_NW, _ROWS_PW, _DIM), jnp.float32)]
    scratch = [
        pltpu.VMEM((_CH,), jnp.int32),
        pltpu.VMEM((_CH,), jnp.int32),
        pltpu.VMEM((_CH,), jnp.int32),
        pltpu.VMEM((_CH, _DIM), jnp.float32),
        pltpu.VMEM_SHARED((_NP, _DIM), jnp.float32),
        pltpu.SemaphoreType.DMA,
        pltpu.SemaphoreType.DMA,
    ]
    if with_deg:
        out_type.append(jax.ShapeDtypeStruct((_NW, _ROWS_PW), jnp.float32))
        scratch += [pltpu.VMEM((_CH,), jnp.float32),
                    pltpu.VMEM_SHARED((_NP,), jnp.float32)]

    @functools.partial(pl.kernel, out_type=out_type, mesh=mesh,
                       scratch_types=scratch)
    def k(tp_h, tn_h, src_h, dst_h, actp_h, actn_h, zeros_h, zeros1_h,
          *rest):
        if with_deg:
            (out_h, deg_out_h, src_v, dst_v, av_v, rows_v, acc_sh, sem,
             sem2, avf_v, deg_sh) = rest
        else:
            (out_h, src_v, dst_v, av_v, rows_v, acc_sh, sem, sem2) = rest
        c = lax.axis_index("c")
        s = lax.axis_index("s")
        wid = c * _NSUB + s
        pltpu.sync_copy(zeros_h, acc_sh.at[pl.ds(s * _ROWS_PW, _ROWS_PW)])
        if with_deg:
            pltpu.sync_copy(zeros1_h,
                            deg_sh.at[pl.ds(s * _ROWS_PW, _ROWS_PW)])
        plsc.subcore_barrier()
        base = s * epw
        trash = jnp.full((16,), _TRASH, jnp.int32)

        def body(t, carry):
            off = base + t * _CH
            pltpu.sync_copy(src_h.at[pl.ds(off, _CH)], src_v)
            pltpu.sync_copy(dst_h.at[pl.ds(off, _CH)], dst_v)

            @pl.when(c == 0)
            def _():
                pltpu.async_copy(actp_h.at[src_v], av_v, sem2).wait()

            @pl.when(c == 1)
            def _():
                pltpu.async_copy(actn_h.at[src_v], av_v, sem2).wait()

            for j in range(_CH // 16):
                sl = pl.ds(j * 16, 16)
                av = av_v[sl]
                dst_v[sl] = jnp.where(av > 0, dst_v[sl], trash)
                if with_deg:
                    avf_v[sl] = av.astype(jnp.float32)

            @pl.when(c == 0)
            def _():
                pltpu.async_copy(tp_h.at[src_v], rows_v, sem).wait()

            @pl.when(c == 1)
            def _():
                pltpu.async_copy(tn_h.at[src_v], rows_v, sem).wait()

            pltpu.sync_copy(rows_v, acc_sh.at[dst_v], add=True)
            if with_deg:
                pltpu.sync_copy(avf_v, deg_sh.at[dst_v], add=True)
            return carry

        lax.fori_loop(0, nch, body, 0)
        plsc.subcore_barrier()
        pltpu.sync_copy(acc_sh.at[pl.ds(s * _ROWS_PW, _ROWS_PW)],
                        out_h.at[wid])
        if with_deg:
            pltpu.sync_copy(deg_sh.at[pl.ds(s * _ROWS_PW, _ROWS_PW)],
                            deg_out_h.at[wid])

    zeros = jnp.zeros((_ROWS_PW, _DIM), jnp.float32)
    zeros1 = jnp.zeros((_ROWS_PW,), jnp.float32)
    if with_deg:
        out, deg = k(table_p, table_n, src, dst, act_p, act_n, zeros,
                     zeros1)
        return (out.reshape(2, _NP, _DIM), deg.reshape(2, _NP))
    (out,) = k(table_p, table_n, src, dst, act_p, act_n, zeros, zeros1)
    return out.reshape(2, _NP, _DIM)


def _sc_gather(table, idx):
    n = idx.shape[0]
    per_w = n // _NW
    nch = per_w // _CH
    mesh = plsc.VectorSubcoreMesh(core_axis_name="c", subcore_axis_name="s")

    @functools.partial(
        pl.kernel,
        out_type=jax.ShapeDtypeStruct((n, _DIM), jnp.float32),
        mesh=mesh,
        scratch_types=[
            pltpu.VMEM((_CH,), jnp.int32),
            pltpu.VMEM((_CH, _DIM), jnp.float32),
            pltpu.SemaphoreType.DMA,
        ],
    )
    def k(table_h, idx_h, out_h, idx_v, rows_v, sem):
        c = lax.axis_index("c")
        s = lax.axis_index("s")
        base = (c * _NSUB + s) * per_w

        def body(t, carry):
            off = base + t * _CH
            pltpu.sync_copy(idx_h.at[pl.ds(off, _CH)], idx_v)
            pltpu.async_copy(table_h.at[idx_v], rows_v, sem).wait()
            pltpu.sync_copy(rows_v, out_h.at[pl.ds(off, _CH)])
            return carry

        lax.fori_loop(0, nch, body, 0)

    return k(table, idx)


_RB = 1024


def _mm_body(x_ref, sp_ref, sn_ref, dp_ref, dn_ref, ws_ref, wn_ref, b_ref,
             hap_ref, han_ref, hb_ref):
    x = x_ref[...]
    ws = ws_ref[...]
    wn = wn_ref[...]
    b = b_ref[...]
    s = lax.dot_general(x, ws, (((1,), (1,)), ((), ())),
                        preferred_element_type=jnp.float32)
    np_ = sp_ref[...] / jnp.maximum(dp_ref[...][:, 0:1], 1.0)
    nn_ = sn_ref[...] / jnp.maximum(dn_ref[...][:, 0:1], 1.0)
    mp = lax.dot_general(np_, wn, (((1,), (1,)), ((), ())),
                         preferred_element_type=jnp.float32)
    mn = lax.dot_general(nn_, wn, (((1,), (1,)), ((), ())),
                         preferred_element_type=jnp.float32)
    hap_ref[...] = jnp.maximum(s + mp + b, 0.0)
    han_ref[...] = jnp.maximum(s + mn + b, 0.0)
    hb_ref[...] = jnp.maximum(s + b, 0.0)


def _tc_sage(x, sum_p, sum_n, deg_p, deg_n, Ws, Wn, b):
    grid = (_NP // _RB,)
    row = pl.BlockSpec((_RB, _DIM), lambda i: (i, 0))
    dcol = pl.BlockSpec((_RB, 16), lambda i: (i, 0))
    full = pl.BlockSpec((_DIM, _DIM), lambda i: (0, 0))
    bspec = pl.BlockSpec((1, _DIM), lambda i: (0, 0))
    out_sh = jax.ShapeDtypeStruct((_NP, _DIM), jnp.float32)
    return pl.pallas_call(
        _mm_body,
        grid=grid,
        in_specs=[row, row, row, dcol, dcol, full, full, bspec],
        out_specs=[row, row, row],
        out_shape=[out_sh, out_sh, out_sh],
    )(x, sum_p, sum_n, deg_p, deg_n, Ws, Wn, b.reshape(1, _DIM))


def _mm2_body(xp_ref, xn_ref, xb_ref, sp_ref, sn_ref, dp_ref, dn_ref,
              ws_ref, wn_ref, b_ref, hap_ref, han_ref, hb_ref):
    ws = ws_ref[...]
    wn = wn_ref[...]
    b = b_ref[...]
    dot = lambda a, w: lax.dot_general(a, w, (((1,), (1,)), ((), ())),
                                       preferred_element_type=jnp.float32)
    np_ = sp_ref[...] / jnp.maximum(dp_ref[...][:, 0:1], 1.0)
    nn_ = sn_ref[...] / jnp.maximum(dn_ref[...][:, 0:1], 1.0)
    hap_ref[...] = jnp.maximum(dot(xp_ref[...], ws) + dot(np_, wn) + b, 0.0)
    han_ref[...] = jnp.maximum(dot(xn_ref[...], ws) + dot(nn_, wn) + b, 0.0)
    hb_ref[...] = jnp.maximum(dot(xb_ref[...], ws) + b, 0.0)


def _tc_sage2(xp, xn, xb, sum_p, sum_n, deg_p, deg_n, Ws, Wn, b):
    grid = (_NP // _RB,)
    row = pl.BlockSpec((_RB, _DIM), lambda i: (i, 0))
    dcol = pl.BlockSpec((_RB, 16), lambda i: (i, 0))
    full = pl.BlockSpec((_DIM, _DIM), lambda i: (0, 0))
    bspec = pl.BlockSpec((1, _DIM), lambda i: (0, 0))
    out_sh = jax.ShapeDtypeStruct((_NP, _DIM), jnp.float32)
    return pl.pallas_call(
        _mm2_body,
        grid=grid,
        in_specs=[row, row, row, row, row, dcol, dcol, full, full, bspec],
        out_specs=[row, row, row],
        out_shape=[out_sh, out_sh, out_sh],
    )(xp, xn, xb, sum_p, sum_n, deg_p, deg_n, Ws, Wn, b.reshape(1, _DIM))


_SB = 128


def _sim_body(g_ref, out_ref):
    h = g_ref[...]
    t = h[:, 0, :]
    l = h[:, 1, :]
    a = h[:, 2:, :]
    nt = jnp.sqrt(jnp.sum(t * t, axis=-1, keepdims=True))
    nl = jnp.sqrt(jnp.sum(l * l, axis=-1, keepdims=True))
    na = jnp.sqrt(jnp.sum(a * a, axis=-1))
    tl = t + l
    ntl = jnp.sqrt(jnp.sum(tl * tl, axis=-1, keepdims=True))
    u = l[:, None, :] + a
    nu = jnp.sqrt(jnp.sum(u * u, axis=-1))
    v = t[:, None, :] + a
    nv = jnp.sqrt(jnp.sum(v * v, axis=-1))
    dt = jnp.sum(t[:, None, :] * u, axis=-1)
    dl = jnp.sum(l[:, None, :] * v, axis=-1)
    da = jnp.sum(a * tl[:, None, :], axis=-1)
    sim_t = jnp.sum(dt / (nt * nu), axis=-1)
    sim_l = jnp.sum(dl / (nl * nv), axis=-1)
    sim_a = jnp.sum(da / (na * ntl), axis=-1)
    out_ref[...] = sim_t + sim_l + sim_a


def _tc_sim(g, nrows):
    grid = (nrows // _SB,)
    return pl.pallas_call(
        _sim_body,
        grid=grid,
        in_specs=[pl.BlockSpec((_SB, _SEQ + 2, _DIM), lambda i: (i, 0, 0))],
        out_specs=pl.BlockSpec((_SB,), lambda i: (i,)),
        out_shape=jax.ShapeDtypeStruct((nrows,), jnp.float32),
    )(g)


def kernel(users, times, locs, app_seq, edge_index, tla_emb,
           Ws1, Wn1, b1, Ws2, Wn2, b2):
    batch = users.shape[0]
    m = batch * (_SEQ + 2)
    nodes_idx = jnp.concatenate(
        [_N_APPS + _N_LOCS + times, _N_APPS + locs, app_seq],
        axis=1).reshape(-1)
    nk = jax.random.key(42)
    nks = jax.random.split(nk, 4)
    neg_t = jax.random.randint(nks[1], (batch, 1), 0, _N_TIMES)
    neg_l = jax.random.randint(nks[2], (batch, 1), 0, _N_LOCS)
    neg_a = jax.random.randint(nks[3], (batch, _SEQ), 0, _N_APPS)
    neg_nodes_idx = jnp.concatenate(
        [_N_APPS + _N_LOCS + neg_t, _N_APPS + neg_l, neg_a],
        axis=1).reshape(-1)

    ar = jnp.arange(m, dtype=jnp.int32)
    lp_pos = jnp.full((_NP,), -1, jnp.int32).at[nodes_idx].max(ar)
    lp_neg = jnp.full((_NP,), -1, jnp.int32).at[neg_nodes_idx].max(ar)
    act_p = (lp_pos >= 0).astype(jnp.int32)
    act_n = (lp_neg >= 0).astype(jnp.int32)
    islast_p = jnp.zeros((m,), jnp.int32).at[
        jnp.where(lp_pos >= 0, lp_pos, m)].set(1, mode="drop")
    islast_n = jnp.zeros((m,), jnp.int32).at[
        jnp.where(lp_neg >= 0, lp_neg, m)].set(1, mode="drop")

    src, dst = edge_index[0], edge_index[1]
    e = src.shape[0]
    ep = -(-e // (_NSUB * _CH)) * (_NSUB * _CH)
    pad = ep - e
    src_p = jnp.concatenate([src.astype(jnp.int32),
                             jnp.zeros((pad,), jnp.int32)])
    dst_p = jnp.concatenate([dst.astype(jnp.int32),
                             jnp.full((pad,), _TRASH, jnp.int32)])

    x_pad = jnp.zeros((_NP, _DIM), jnp.float32).at[:_N_NODES].set(tla_emb)

    acc_a, deg = _sc_aggregate(x_pad, x_pad, src_p, dst_p,
                               act_p, act_n, True)
    sum1_p, sum1_n = acc_a[0], acc_a[1]
    dg_p = jnp.broadcast_to(deg[0][:, None], (_NP, 16))
    dg_n = jnp.broadcast_to(deg[1][:, None], (_NP, 16))

    h1a_p, h1a_n, h1b = _tc_sage(x_pad, sum1_p, sum1_n, dg_p, dg_n,
                                 Ws1, Wn1, b1)

    acc_b = _sc_aggregate(h1a_p, h1a_n, src_p, dst_p, act_p, act_n, False)
    sum2_p, sum2_n = acc_b[0], acc_b[1]

    h2a_p, h2a_n, h2b = _tc_sage2(h1a_p, h1a_n, h1b, sum2_p, sum2_n,
                                  dg_p, dg_n, Ws2, Wn2, b2)

    tf = jnp.concatenate([h2a_p, h2a_n, h2b], axis=0)
    idx_pos = jnp.where(islast_p == 1, nodes_idx, nodes_idx + 2 * _NP)
    idx_neg = jnp.where(islast_n == 1, neg_nodes_idx + _NP,
                        neg_nodes_idx + 2 * _NP)
    idx_all = jnp.concatenate([idx_pos, idx_neg]).astype(jnp.int32)
    g = _sc_gather(tf, idx_all)
    loss = _tc_sim(g.reshape(2 * batch, _SEQ + 2, _DIM), 2 * batch)
    return loss[:batch], loss[batch:]


# speedup vs baseline: 9.0653x; 1.2690x over previous
import functools

import jax
import jax.numpy as jnp
from jax import lax
from jax.experimental import pallas as pl
from jax.experimental.pallas import tpu as pltpu
from jax.experimental.pallas import tpu_sc as plsc

_N_TIMES = 1000
_N_LOCS = 1000
_N_APPS = 8000
_N_NODES = 10000
_DIM = 128
_SEQ = 20

_NP = 10240
_TRASH = 10100
_CH = 128
_NSUB = 16
_NW = 32
_ROWS_PW = _NP // _NSUB


def _sc_aggregate(table_p, table_n, src, dst, act_p, act_n, with_deg):
    ep = src.shape[0]
    epw = ep // _NSUB
    nch = epw // _CH
    assert nch % 2 == 0
    mesh = plsc.VectorSubcoreMesh(core_axis_name="c", subcore_axis_name="s")

    out_type = [jax.ShapeDtypeStruct((_NW, _ROWS_PW, _DIM), jnp.float32)]
    scratch = [
        pltpu.VMEM((_CH,), jnp.int32),
        pltpu.VMEM((_CH,), jnp.int32),
        pltpu.VMEM((_CH,), jnp.int32),
        pltpu.VMEM((_CH,), jnp.int32),
        pltpu.VMEM((_CH,), jnp.int32),
        pltpu.VMEM((_CH,), jnp.int32),
        pltpu.VMEM((_CH, _DIM), jnp.float32),
        pltpu.VMEM((_CH, _DIM), jnp.float32),
        pltpu.VMEM_SHARED((_NP, _DIM), jnp.float32),
        pltpu.SemaphoreType.DMA,
        pltpu.SemaphoreType.DMA,
        pltpu.SemaphoreType.DMA,
        pltpu.SemaphoreType.DMA,
        pltpu.SemaphoreType.DMA,
        pltpu.SemaphoreType.DMA,
    ]
    if with_deg:
        out_type.append(jax.ShapeDtypeStruct((_NW, _ROWS_PW), jnp.float32))
        scratch += [pltpu.VMEM((_CH,), jnp.float32),
                    pltpu.VMEM((_CH,), jnp.float32),
                    pltpu.VMEM_SHARED((_NP,), jnp.float32)]

    @functools.partial(pl.kernel, out_type=out_type, mesh=mesh,
                       scratch_types=scratch)
    def k(tp_h, tn_h, src_h, dst_h, actp_h, actn_h, zeros_h, zeros1_h,
          *rest):
        if with_deg:
            (out_h, deg_out_h, sva, svb, dva, dvb, ava, avb, rva, rvb,
             acc_sh, s_sta, s_stb, s_aa, s_ab, s_ra, s_rb, fva, fvb,
             deg_sh) = rest
        else:
            (out_h, sva, svb, dva, dvb, ava, avb, rva, rvb, acc_sh,
             s_sta, s_stb, s_aa, s_ab, s_ra, s_rb) = rest
            fva = fvb = deg_sh = None
        c = lax.axis_index("c")
        s = lax.axis_index("s")
        wid = c * _NSUB + s
        pltpu.sync_copy(zeros_h, acc_sh.at[pl.ds(s * _ROWS_PW, _ROWS_PW)])
        if with_deg:
            pltpu.sync_copy(zeros1_h,
                            deg_sh.at[pl.ds(s * _ROWS_PW, _ROWS_PW)])
        plsc.subcore_barrier()
        base = s * epw
        trash = jnp.full((16,), _TRASH, jnp.int32)

        def stage(off, sv, dv, sem):
            pltpu.async_copy(src_h.at[pl.ds(off, _CH)], sv, sem)
            pltpu.async_copy(dst_h.at[pl.ds(off, _CH)], dv, sem)

        def stage_wait(off, sv, dv, sem):
            pltpu.make_async_copy(src_h.at[pl.ds(off, _CH)], sv, sem).wait()
            pltpu.make_async_copy(dst_h.at[pl.ds(off, _CH)], dv, sem).wait()

        def issue_gathers(sv, av, rv, s_a, s_r):
            @pl.when(c == 0)
            def _():
                pltpu.async_copy(actp_h.at[sv], av, s_a)
                pltpu.async_copy(tp_h.at[sv], rv, s_r)

            @pl.when(c == 1)
            def _():
                pltpu.async_copy(actn_h.at[sv], av, s_a)
                pltpu.async_copy(tn_h.at[sv], rv, s_r)

        def finish(sv, dv, av, rv, fv, s_a, s_r):
            pltpu.make_async_copy(actp_h.at[sv], av, s_a).wait()
            for j in range(_CH // 16):
                sl = pl.ds(j * 16, 16)
                a16 = av[sl]
                dv[sl] = jnp.where(a16 > 0, dv[sl], trash)
                if with_deg:
                    fv[sl] = a16.astype(jnp.float32)
            pltpu.make_async_copy(tp_h.at[sv], rv, s_r).wait()
            pltpu.sync_copy(rv, acc_sh.at[dv], add=True)
            if with_deg:
                pltpu.sync_copy(fv, deg_sh.at[dv], add=True)

        def body(t, carry):
            off_a = base + (2 * t) * _CH
            off_b = off_a + _CH
            stage(off_a, sva, dva, s_sta)
            stage(off_b, svb, dvb, s_stb)
            stage_wait(off_a, sva, dva, s_sta)
            issue_gathers(sva, ava, rva, s_aa, s_ra)
            stage_wait(off_b, svb, dvb, s_stb)
            issue_gathers(svb, avb, rvb, s_ab, s_rb)
            finish(sva, dva, ava, rva, fva, s_aa, s_ra)
            finish(svb, dvb, avb, rvb, fvb, s_ab, s_rb)
            return carry

        lax.fori_loop(0, nch // 2, body, 0)
        plsc.subcore_barrier()
        pltpu.sync_copy(acc_sh.at[pl.ds(s * _ROWS_PW, _ROWS_PW)],
                        out_h.at[wid])
        if with_deg:
            pltpu.sync_copy(deg_sh.at[pl.ds(s * _ROWS_PW, _ROWS_PW)],
                            deg_out_h.at[wid])

    zeros = jnp.zeros((_ROWS_PW, _DIM), jnp.float32)
    zeros1 = jnp.zeros((_ROWS_PW,), jnp.float32)
    if with_deg:
        out, deg = k(table_p, table_n, src, dst, act_p, act_n, zeros,
                     zeros1)
        return (out.reshape(2, _NP, _DIM), deg.reshape(2, _NP))
    (out,) = k(table_p, table_n, src, dst, act_p, act_n, zeros, zeros1)
    return out.reshape(2, _NP, _DIM)


def _sc_gather(table, idx):
    n = idx.shape[0]
    per_w = n // _NW
    nch = per_w // _CH
    assert nch % 2 == 0
    mesh = plsc.VectorSubcoreMesh(core_axis_name="c", subcore_axis_name="s")

    @functools.partial(
        pl.kernel,
        out_type=jax.ShapeDtypeStruct((n, _DIM), jnp.float32),
        mesh=mesh,
        scratch_types=[
            pltpu.VMEM((_CH,), jnp.int32),
            pltpu.VMEM((_CH,), jnp.int32),
            pltpu.VMEM((_CH, _DIM), jnp.float32),
            pltpu.VMEM((_CH, _DIM), jnp.float32),
            pltpu.SemaphoreType.DMA,
            pltpu.SemaphoreType.DMA,
            pltpu.SemaphoreType.DMA,
            pltpu.SemaphoreType.DMA,
        ],
    )
    def k(table_h, idx_h, out_h, iva, ivb, rva, rvb, s_ia, s_ib, s_ra,
          s_rb):
        c = lax.axis_index("c")
        s = lax.axis_index("s")
        base = (c * _NSUB + s) * per_w

        def body(t, carry):
            off_a = base + (2 * t) * _CH
            off_b = off_a + _CH
            pltpu.async_copy(idx_h.at[pl.ds(off_a, _CH)], iva, s_ia)
            pltpu.async_copy(idx_h.at[pl.ds(off_b, _CH)], ivb, s_ib)
            pltpu.make_async_copy(idx_h.at[pl.ds(off_a, _CH)], iva,
                                  s_ia).wait()
            pltpu.async_copy(table_h.at[iva], rva, s_ra)
            pltpu.make_async_copy(idx_h.at[pl.ds(off_b, _CH)], ivb,
                                  s_ib).wait()
            pltpu.async_copy(table_h.at[ivb], rvb, s_rb)
            pltpu.make_async_copy(table_h.at[iva], rva, s_ra).wait()
            pltpu.sync_copy(rva, out_h.at[pl.ds(off_a, _CH)])
            pltpu.make_async_copy(table_h.at[ivb], rvb, s_rb).wait()
            pltpu.sync_copy(rvb, out_h.at[pl.ds(off_b, _CH)])
            return carry

        lax.fori_loop(0, nch // 2, body, 0)

    return k(table, idx)


_RB = 1024


def _mm_body(x_ref, sp_ref, sn_ref, dp_ref, dn_ref, ws_ref, wn_ref, b_ref,
             hap_ref, han_ref, hb_ref):
    x = x_ref[...]
    ws = ws_ref[...]
    wn = wn_ref[...]
    b = b_ref[...]
    s = lax.dot_general(x, ws, (((1,), (1,)), ((), ())),
                        preferred_element_type=jnp.float32)
    np_ = sp_ref[...] / jnp.maximum(dp_ref[...][:, 0:1], 1.0)
    nn_ = sn_ref[...] / jnp.maximum(dn_ref[...][:, 0:1], 1.0)
    mp = lax.dot_general(np_, wn, (((1,), (1,)), ((), ())),
                         preferred_element_type=jnp.float32)
    mn = lax.dot_general(nn_, wn, (((1,), (1,)), ((), ())),
                         preferred_element_type=jnp.float32)
    hap_ref[...] = jnp.maximum(s + mp + b, 0.0)
    han_ref[...] = jnp.maximum(s + mn + b, 0.0)
    hb_ref[...] = jnp.maximum(s + b, 0.0)


def _tc_sage(x, sum_p, sum_n, deg_p, deg_n, Ws, Wn, b):
    grid = (_NP // _RB,)
    row = pl.BlockSpec((_RB, _DIM), lambda i: (i, 0))
    dcol = pl.BlockSpec((_RB, 16), lambda i: (i, 0))
    full = pl.BlockSpec((_DIM, _DIM), lambda i: (0, 0))
    bspec = pl.BlockSpec((1, _DIM), lambda i: (0, 0))
    out_sh = jax.ShapeDtypeStruct((_NP, _DIM), jnp.float32)
    return pl.pallas_call(
        _mm_body,
        grid=grid,
        in_specs=[row, row, row, dcol, dcol, full, full, bspec],
        out_specs=[row, row, row],
        out_shape=[out_sh, out_sh, out_sh],
    )(x, sum_p, sum_n, deg_p, deg_n, Ws, Wn, b.reshape(1, _DIM))


def _mm2_body(xp_ref, xn_ref, xb_ref, sp_ref, sn_ref, dp_ref, dn_ref,
              ws_ref, wn_ref, b_ref, hap_ref, han_ref, hb_ref):
    ws = ws_ref[...]
    wn = wn_ref[...]
    b = b_ref[...]
    dot = lambda a, w: lax.dot_general(a, w, (((1,), (1,)), ((), ())),
                                       preferred_element_type=jnp.float32)
    np_ = sp_ref[...] / jnp.maximum(dp_ref[...][:, 0:1], 1.0)
    nn_ = sn_ref[...] / jnp.maximum(dn_ref[...][:, 0:1], 1.0)
    hap_ref[...] = jnp.maximum(dot(xp_ref[...], ws) + dot(np_, wn) + b, 0.0)
    han_ref[...] = jnp.maximum(dot(xn_ref[...], ws) + dot(nn_, wn) + b, 0.0)
    hb_ref[...] = jnp.maximum(dot(xb_ref[...], ws) + b, 0.0)


def _tc_sage2(xp, xn, xb, sum_p, sum_n, deg_p, deg_n, Ws, Wn, b):
    grid = (_NP // _RB,)
    row = pl.BlockSpec((_RB, _DIM), lambda i: (i, 0))
    dcol = pl.BlockSpec((_RB, 16), lambda i: (i, 0))
    full = pl.BlockSpec((_DIM, _DIM), lambda i: (0, 0))
    bspec = pl.BlockSpec((1, _DIM), lambda i: (0, 0))
    out_sh = jax.ShapeDtypeStruct((_NP, _DIM), jnp.float32)
    return pl.pallas_call(
        _mm2_body,
        grid=grid,
        in_specs=[row, row, row, row, row, dcol, dcol, full, full, bspec],
        out_specs=[row, row, row],
        out_shape=[out_sh, out_sh, out_sh],
    )(xp, xn, xb, sum_p, sum_n, deg_p, deg_n, Ws, Wn, b.reshape(1, _DIM))


_SB = 128


def _sim_body(g_ref, out_ref):
    h = g_ref[...]
    t = h[:, 0, :]
    l = h[:, 1, :]
    a = h[:, 2:, :]
    nt2 = jnp.sum(t * t, axis=-1, keepdims=True)
    nl2 = jnp.sum(l * l, axis=-1, keepdims=True)
    stl = jnp.sum(t * l, axis=-1, keepdims=True)
    na2 = jnp.sum(a * a, axis=-1)
    p = jnp.sum(t[:, None, :] * a, axis=-1)
    q = jnp.sum(l[:, None, :] * a, axis=-1)
    nt = jnp.sqrt(nt2)
    nl = jnp.sqrt(nl2)
    ntl = jnp.sqrt(nt2 + 2.0 * stl + nl2)
    nu = jnp.sqrt(nl2 + 2.0 * q + na2)
    nv = jnp.sqrt(nt2 + 2.0 * p + na2)
    na = jnp.sqrt(na2)
    sim_t = jnp.sum((stl + p) / (nt * nu), axis=-1)
    sim_l = jnp.sum((stl + q) / (nl * nv), axis=-1)
    sim_a = jnp.sum((p + q) / (na * ntl), axis=-1)
    out_ref[...] = sim_t + sim_l + sim_a


def _tc_sim(g, nrows):
    grid = (nrows // _SB,)
    return pl.pallas_call(
        _sim_body,
        grid=grid,
        in_specs=[pl.BlockSpec((_SB, _SEQ + 2, _DIM), lambda i: (i, 0, 0))],
        out_specs=pl.BlockSpec((_SB,), lambda i: (i,)),
        out_shape=jax.ShapeDtypeStruct((nrows,), jnp.float32),
    )(g)


def kernel(users, times, locs, app_seq, edge_index, tla_emb,
           Ws1, Wn1, b1, Ws2, Wn2, b2):
    batch = users.shape[0]
    m = batch * (_SEQ + 2)
    nodes_idx = jnp.concatenate(
        [_N_APPS + _N_LOCS + times, _N_APPS + locs, app_seq],
        axis=1).reshape(-1)
    nk = jax.random.key(42)
    nks = jax.random.split(nk, 4)
    neg_t = jax.random.randint(nks[1], (batch, 1), 0, _N_TIMES)
    neg_l = jax.random.randint(nks[2], (batch, 1), 0, _N_LOCS)
    neg_a = jax.random.randint(nks[3], (batch, _SEQ), 0, _N_APPS)
    neg_nodes_idx = jnp.concatenate(
        [_N_APPS + _N_LOCS + neg_t, _N_APPS + neg_l, neg_a],
        axis=1).reshape(-1)

    ar = jnp.arange(m, dtype=jnp.int32)
    lp_pos = jnp.full((_NP,), -1, jnp.int32).at[nodes_idx].max(ar)
    lp_neg = jnp.full((_NP,), -1, jnp.int32).at[neg_nodes_idx].max(ar)
    act_p = (lp_pos >= 0).astype(jnp.int32)
    act_n = (lp_neg >= 0).astype(jnp.int32)
    islast_p = jnp.zeros((m,), jnp.int32).at[
        jnp.where(lp_pos >= 0, lp_pos, m)].set(1, mode="drop")
    islast_n = jnp.zeros((m,), jnp.int32).at[
        jnp.where(lp_neg >= 0, lp_neg, m)].set(1, mode="drop")

    src, dst = edge_index[0], edge_index[1]
    e = src.shape[0]
    ep = -(-e // (_NSUB * _CH * 2)) * (_NSUB * _CH * 2)
    pad = ep - e
    src_p = jnp.concatenate([src.astype(jnp.int32),
                             jnp.zeros((pad,), jnp.int32)])
    dst_p = jnp.concatenate([dst.astype(jnp.int32),
                             jnp.full((pad,), _TRASH, jnp.int32)])

    x_pad = jnp.zeros((_NP, _DIM), jnp.float32).at[:_N_NODES].set(tla_emb)

    acc_a, deg = _sc_aggregate(x_pad, x_pad, src_p, dst_p,
                               act_p, act_n, True)
    sum1_p, sum1_n = acc_a[0], acc_a[1]
    dg_p = jnp.broadcast_to(deg[0][:, None], (_NP, 16))
    dg_n = jnp.broadcast_to(deg[1][:, None], (_NP, 16))

    h1a_p, h1a_n, h1b = _tc_sage(x_pad, sum1_p, sum1_n, dg_p, dg_n,
                                 Ws1, Wn1, b1)

    acc_b = _sc_aggregate(h1a_p, h1a_n, src_p, dst_p, act_p, act_n, False)
    sum2_p, sum2_n = acc_b[0], acc_b[1]

    h2a_p, h2a_n, h2b = _tc_sage2(h1a_p, h1a_n, h1b, sum2_p, sum2_n,
                                  dg_p, dg_n, Ws2, Wn2, b2)

    tf = jnp.concatenate([h2a_p, h2a_n, h2b], axis=0)
    idx_pos = jnp.where(islast_p == 1, nodes_idx, nodes_idx + 2 * _NP)
    idx_neg = jnp.where(islast_n == 1, neg_nodes_idx + _NP,
                        neg_nodes_idx + 2 * _NP)
    idx_all = jnp.concatenate([idx_pos, idx_neg]).astype(jnp.int32)
    g = _sc_gather(tf, idx_all)
    loss = _tc_sim(g.reshape(2 * batch, _SEQ + 2, _DIM), 2 * batch)
    return loss[:batch], loss[batch:]
